# Initial kernel scaffold; baseline (speedup 1.0000x reference)
import jax
import jax.numpy as jnp
from jax.experimental import pallas as pl


def kernel(memory, idx, values):
    raise NotImplementedError("write your pallas kernel here")


# speedup vs baseline: 1.8761x; 1.8761x over previous
import functools

import jax
import jax.numpy as jnp
from jax import lax
from jax.experimental import pallas as pl
from jax.experimental.pallas import tpu as pltpu
from jax.experimental.pallas import tpu_sc as plsc

M = 2097152
N = 1048576
F = 16
NB = 32
RB = M // NB
NT = 32
TSL = N // NT
CAP = 64
BUFW = 80
BIN_CAP = N + NB * NT * CAP

_cp = pltpu.CompilerParams(needs_layout_passes=False, use_tc_tiling_on_sc=False)

_GDN = lax.GatherDimensionNumbers(
    offset_dims=(), collapsed_slice_dims=(0,), start_index_map=(0,))


def _dgat(x, i):
    return lax.gather(x, i.reshape(16, 1), dimension_numbers=_GDN,
                      slice_sizes=(1,),
                      mode=lax.GatherScatterMode.PROMISE_IN_BOUNDS)


def _iota():
    return lax.iota(jnp.int32, 16)


def _extract32(ref, b):
    i = _iota()
    lo = ref[pl.ds(0, 16)]
    hi = ref[pl.ds(16, 16)]
    v = jnp.where(i == b, lo, 0) + jnp.where(i + 16 == b, hi, 0)
    return jnp.sum(v)


def _adjust32(ref, b, delta):
    i = _iota()
    ref[pl.ds(0, 16)] = ref[pl.ds(0, 16)] + jnp.where(i == b, delta, 0)
    ref[pl.ds(16, 16)] = ref[pl.ds(16, 16)] + jnp.where(i + 16 == b, delta, 0)


def _k1_body(idx_hbm, cnt_hbm, idx_v, hist_v):
    cid = lax.axis_index("c")
    sid = lax.axis_index("s")
    wid = sid * 2 + cid
    pltpu.sync_copy(idx_hbm.at[pl.ds(wid * TSL, TSL)], idx_v)
    z = jnp.zeros((16,), jnp.int32)
    hist_v[pl.ds(0, 16)] = z
    hist_v[pl.ds(16, 16)] = z
    ones = jnp.ones((16,), jnp.int32)

    def body(i, carry):
        x = idx_v[pl.ds(i * 16, 16)]
        plsc.addupdate_scatter(hist_v, [lax.shift_right_logical(x, 16)], ones)
        return carry

    lax.fori_loop(0, TSL // 16, body, 0)
    pltpu.sync_copy(hist_v, cnt_hbm.at[wid])


def _k2_body(idx_hbm, gstart_hbm, binned_hbm, idx_v, buf_v, cnt_v, gcur_v):
    cid = lax.axis_index("c")
    sid = lax.axis_index("s")
    wid = sid * 2 + cid
    i16 = _iota()
    pltpu.sync_copy(idx_hbm.at[pl.ds(wid * TSL, TSL)], idx_v)
    pltpu.sync_copy(gstart_hbm.at[wid], gcur_v)
    z = jnp.zeros((16,), jnp.int32)
    cnt_v[pl.ds(0, 16)] = z
    cnt_v[pl.ds(16, 16)] = z
    ones = jnp.ones((16,), jnp.int32)

    def flush(b):
        ofs = _extract32(gcur_v, b)
        dst = pl.multiple_of(2 * ofs, 2 * CAP)
        pltpu.sync_copy(buf_v.at[pl.ds(b * 2 * BUFW, 2 * CAP)],
                        binned_hbm.at[pl.ds(dst, 2 * CAP)])
        r0 = buf_v[pl.ds(b * 2 * BUFW + 2 * CAP, 16)]
        r1 = buf_v[pl.ds(b * 2 * BUFW + 2 * CAP + 16, 16)]
        buf_v[pl.ds(b * 2 * BUFW, 16)] = r0
        buf_v[pl.ds(b * 2 * BUFW + 16, 16)] = r1
        _adjust32(gcur_v, b, CAP)
        _adjust32(cnt_v, b, -CAP)

    def body(i, carry):
        x = idx_v[pl.ds(i * 16, 16)]
        posg = i16 + (wid * TSL + 16 * i)
        sk, perm = plsc.sort_key_val(x, i16)
        pos_s = _dgat(posg, perm)
        binv = lax.shift_right_logical(sk, 16)
        prev = _dgat(binv, jnp.maximum(i16 - 1, 0))
        is_new = jnp.logical_or(i16 == 0, binv != prev)
        start = plsc.cummax(jnp.where(is_new, i16, 0))
        rank = i16 - start
        cur = plsc.load_gather(cnt_v, [binv])
        slot = cur + rank
        fb = binv * (2 * BUFW) + 2 * slot
        plsc.store_scatter(buf_v, [fb], pos_s)
        plsc.store_scatter(buf_v, [fb + 1], sk)
        plsc.addupdate_scatter(cnt_v, [binv], ones)
        act0 = slot + 1 >= CAP
        fmax0 = jnp.max(jnp.where(act0, binv, -1))

        def w_cond(c):
            return c[0] >= 0

        def w_body(c):
            b, act = c
            flush(b)
            act2 = jnp.logical_and(act, binv != b)
            return jnp.max(jnp.where(act2, binv, -1)), act2

        lax.while_loop(w_cond, w_body, (fmax0, act0))
        return carry

    lax.fori_loop(0, TSL // 16, body, 0)

    sign = jnp.int32(-2147483648)
    for b in range(NB):
        cb = _extract32(cnt_v, b)

        @pl.when(cb > 0)
        def _(b=b, cb=cb):
            eoff = i16 * 0 + (b * 2 * BUFW + 2 * (cb - 1))
            lp = plsc.load_gather(buf_v, [eoff])
            li = lax.bitwise_or(plsc.load_gather(buf_v, [eoff + 1]), sign)
            for g in range(2 * CAP // 16):
                sl = buf_v[pl.ds(b * 2 * BUFW + 16 * g, 16)]
                gpos = i16 + 16 * g
                ent = lax.shift_right_logical(gpos, 1)
                padv = jnp.where(lax.rem(gpos, 2) == 1, li, lp)
                buf_v[pl.ds(b * 2 * BUFW + 16 * g, 16)] = jnp.where(
                    ent >= cb, padv, sl)
            ofs = _extract32(gcur_v, b)
            dst = pl.multiple_of(2 * ofs, 2 * CAP)
            pltpu.sync_copy(buf_v.at[pl.ds(b * 2 * BUFW, 2 * CAP)],
                            binned_hbm.at[pl.ds(dst, 2 * CAP)])


def _k3_body(mem_hbm, val_hbm, binned_hbm, starts_hbm, out_hbm,
             sv, win_v, pos_v, lidx_v, pos_t, lidx_t, vrows_v, orows_v,
             accum, sem):
    cid = lax.axis_index("c")
    sid = lax.axis_index("s")
    i16 = _iota()
    pltpu.sync_copy(starts_hbm, sv)
    s0 = sv[pl.ds(0, 16)]
    s1 = sv[pl.ds(16, 16)]
    s2 = sv[pl.ds(32, 16)]

    def sget(k):
        v = (jnp.where(i16 == k, s0, 0) + jnp.where(i16 + 16 == k, s1, 0)
             + jnp.where(i16 + 32 == k, s2, 0))
        return jnp.sum(v)

    def stage_entries(eo, nent, base, for_accum):
        src = pl.multiple_of(2 * eo, 128)
        pltpu.sync_copy(binned_hbm.at[pl.ds(src, 2 * nent)],
                        win_v.at[pl.ds(0, 2 * nent)])
        mask31 = jnp.int32(2147483647)
        for g in range(nent // 16):
            e2 = 32 * g + 2 * i16
            p = plsc.load_gather(win_v, [e2])
            q = plsc.load_gather(win_v, [e2 + 1])
            l = lax.bitwise_and(q, mask31) - base
            if for_accum:
                l = jnp.where(q >= 0, l, RB)
            if nent == 64:
                pos_t[pl.ds(16 * g, 16)] = p
                lidx_t[pl.ds(16 * g, 16)] = l
            else:
                pos_v[g // 8, pl.ds(16 * (g % 8), 16)] = p
                lidx_v[g // 8, pl.ds(16 * (g % 8), 16)] = l

    def idx_refs(nent):
        if nent == 64:
            return [(pos_t, lidx_t, 0)]
        return [(pos_v.at[j], lidx_v.at[j], 128 * j) for j in range(4)]

    def accum_chunk(eo, nent, base):
        stage_entries(eo, nent, base, True)
        sub = min(nent, 128)
        cps = [pltpu.async_copy(
            val_hbm.at[pr], vrows_v.at[pl.ds(o, sub)], sem)
            for pr, _, o in idx_refs(nent)]
        for c in cps:
            c.wait()
        for _, lr, o in idx_refs(nent):
            pltpu.sync_copy(vrows_v.at[pl.ds(o, sub)], accum.at[lr], add=True)

    def out_chunk(eo, nent, base):
        stage_entries(eo, nent, base, False)
        sub = min(nent, 128)
        cps = [pltpu.async_copy(
            accum.at[lr], orows_v.at[pl.ds(o, sub)], sem)
            for _, lr, o in idx_refs(nent)]
        for c in cps:
            c.wait()
        cps = [pltpu.async_copy(
            orows_v.at[pl.ds(o, sub)], out_hbm.at[pr], sem)
            for pr, _, o in idx_refs(nent)]
        for c in cps:
            c.wait()

    def sweep(lo, hi, S, base, chunk):
        def m_cond(u):
            return u + 8 <= hi

        def m_body(u):
            chunk(S + 64 * u, 512, base)
            return u + 8

        u = lax.while_loop(m_cond, m_body, lo)

        def t_cond(u):
            return u < hi

        def t_body(u):
            chunk(S + 64 * u, 64, base)
            return u + 1

        lax.while_loop(t_cond, t_body, u)

    def bin_body(h, carry):
        b = cid * 16 + h
        S = sget(b)
        E = sget(b + 1)
        base = b * RB
        pltpu.sync_copy(mem_hbm.at[pl.ds(base + sid * (RB // 16), RB // 16)],
                        accum.at[pl.ds(sid * (RB // 16), RB // 16)])
        plsc.subcore_barrier()
        nb64 = lax.shift_right_logical(E - S, 6)
        share = lax.shift_right_logical(nb64 + 15, 4)
        lo = jnp.minimum(sid * share, nb64)
        hi = jnp.minimum(lo + share, nb64)
        sweep(lo, hi, S, base, accum_chunk)
        plsc.subcore_barrier()
        sweep(lo, hi, S, base, out_chunk)
        plsc.subcore_barrier()
        return carry

    lax.fori_loop(0, 16, bin_body, 0)


@functools.cache
def _get_kernels():
    mesh = plsc.VectorSubcoreMesh(core_axis_name="c", subcore_axis_name="s")
    k1 = functools.partial(
        pl.kernel, mesh=mesh, compiler_params=_cp,
        out_type=jax.ShapeDtypeStruct((NT, NB), jnp.int32),
        scratch_types=[pltpu.VMEM((TSL,), jnp.int32),
                       pltpu.VMEM((NB,), jnp.int32)],
    )(_k1_body)
    k2 = functools.partial(
        pl.kernel, mesh=mesh, compiler_params=_cp,
        out_type=jax.ShapeDtypeStruct((2 * BIN_CAP,), jnp.int32),
        scratch_types=[pltpu.VMEM((TSL,), jnp.int32),
                       pltpu.VMEM((NB * 2 * BUFW,), jnp.int32),
                       pltpu.VMEM((NB,), jnp.int32),
                       pltpu.VMEM((NB,), jnp.int32)],
    )(_k2_body)
    k3 = functools.partial(
        pl.kernel, mesh=mesh, compiler_params=_cp,
        out_type=jax.ShapeDtypeStruct((N, F), jnp.float32),
        scratch_types=[pltpu.VMEM((48,), jnp.int32),
                       pltpu.VMEM((1024,), jnp.int32),
                       pltpu.VMEM((4, 128), jnp.int32),
                       pltpu.VMEM((4, 128), jnp.int32),
                       pltpu.VMEM((64,), jnp.int32),
                       pltpu.VMEM((64,), jnp.int32),
                       pltpu.VMEM((512, F), jnp.float32),
                       pltpu.VMEM((512, F), jnp.float32),
                       pltpu.VMEM_SHARED((RB + 16, F), jnp.float32),
                       pltpu.SemaphoreType.DMA],
    )(_k3_body)
    return k1, k2, k3


def kernel(memory, idx, values):
    k1, k2, k3 = _get_kernels()
    counts = k1(idx)
    padded = ((counts + (CAP - 1)) // CAP) * CAP
    seg = padded.T.reshape(-1)
    ends = jnp.cumsum(seg)
    starts_flat = (ends - seg).astype(jnp.int32)
    gstart = starts_flat.reshape(NB, NT).T.astype(jnp.int32)
    bin_starts = jnp.concatenate(
        [starts_flat.reshape(NB, NT)[:, 0], ends[-1:].astype(jnp.int32)])
    starts48 = jnp.zeros((48,), jnp.int32).at[0:33].set(bin_starts)
    binned = k2(idx, gstart)
    out = k3(memory, values, binned, starts48)
    return out


# speedup vs baseline: 1.9100x; 1.0181x over previous
import functools

import jax
import jax.numpy as jnp
from jax import lax
from jax.experimental import pallas as pl
from jax.experimental.pallas import tpu as pltpu
from jax.experimental.pallas import tpu_sc as plsc

M = 2097152
N = 1048576
F = 16
NB = 32
RB = M // NB
NT = 32
TSL = N // NT
CAP = 64
BUFW = 80
BIN_CAP = N + NB * NT * CAP

_cp = pltpu.CompilerParams(needs_layout_passes=False, use_tc_tiling_on_sc=False)

_GDN = lax.GatherDimensionNumbers(
    offset_dims=(), collapsed_slice_dims=(0,), start_index_map=(0,))


def _dgat(x, i):
    return lax.gather(x, i.reshape(16, 1), dimension_numbers=_GDN,
                      slice_sizes=(1,),
                      mode=lax.GatherScatterMode.PROMISE_IN_BOUNDS)


def _iota():
    return lax.iota(jnp.int32, 16)


def _extract32(ref, b):
    i = _iota()
    lo = ref[pl.ds(0, 16)]
    hi = ref[pl.ds(16, 16)]
    v = jnp.where(i == b, lo, 0) + jnp.where(i + 16 == b, hi, 0)
    return jnp.sum(v)


def _adjust32(ref, b, delta):
    i = _iota()
    ref[pl.ds(0, 16)] = ref[pl.ds(0, 16)] + jnp.where(i == b, delta, 0)
    ref[pl.ds(16, 16)] = ref[pl.ds(16, 16)] + jnp.where(i + 16 == b, delta, 0)


def _k1_body(idx_hbm, cnt_hbm, idx_v, hist_v):
    cid = lax.axis_index("c")
    sid = lax.axis_index("s")
    wid = sid * 2 + cid
    pltpu.sync_copy(idx_hbm.at[pl.ds(wid * TSL, TSL)], idx_v)
    z = jnp.zeros((16,), jnp.int32)
    hist_v[pl.ds(0, 16)] = z
    hist_v[pl.ds(16, 16)] = z
    ones = jnp.ones((16,), jnp.int32)

    def body(i, carry):
        x = idx_v[pl.ds(i * 16, 16)]
        plsc.addupdate_scatter(hist_v, [lax.shift_right_logical(x, 16)], ones)
        return carry

    lax.fori_loop(0, TSL // 16, body, 0)
    pltpu.sync_copy(hist_v, cnt_hbm.at[wid])


def _k2_body(idx_hbm, gstart_hbm, binned_hbm, idx_v, buf_v, cnt_v, gcur_v):
    cid = lax.axis_index("c")
    sid = lax.axis_index("s")
    wid = sid * 2 + cid
    i16 = _iota()
    pltpu.sync_copy(idx_hbm.at[pl.ds(wid * TSL, TSL)], idx_v)
    pltpu.sync_copy(gstart_hbm.at[wid], gcur_v)
    z = jnp.zeros((16,), jnp.int32)
    cnt_v[pl.ds(0, 16)] = z
    cnt_v[pl.ds(16, 16)] = z
    ones = jnp.ones((16,), jnp.int32)

    def flush(b):
        ofs = _extract32(gcur_v, b)
        dst = pl.multiple_of(2 * ofs, 2 * CAP)
        pltpu.sync_copy(buf_v.at[pl.ds(b * 2 * BUFW, 2 * CAP)],
                        binned_hbm.at[pl.ds(dst, 2 * CAP)])
        r0 = buf_v[pl.ds(b * 2 * BUFW + 2 * CAP, 16)]
        r1 = buf_v[pl.ds(b * 2 * BUFW + 2 * CAP + 16, 16)]
        buf_v[pl.ds(b * 2 * BUFW, 16)] = r0
        buf_v[pl.ds(b * 2 * BUFW + 16, 16)] = r1
        _adjust32(gcur_v, b, CAP)
        _adjust32(cnt_v, b, -CAP)

    def body(i, carry):
        x = idx_v[pl.ds(i * 16, 16)]
        posg = i16 + (wid * TSL + 16 * i)
        sk, perm = plsc.sort_key_val(x, i16)
        pos_s = _dgat(posg, perm)
        binv = lax.shift_right_logical(sk, 16)
        prev = _dgat(binv, jnp.maximum(i16 - 1, 0))
        is_new = jnp.logical_or(i16 == 0, binv != prev)
        start = plsc.cummax(jnp.where(is_new, i16, 0))
        rank = i16 - start
        cur = plsc.load_gather(cnt_v, [binv])
        slot = cur + rank
        fb = binv * (2 * BUFW) + 2 * slot
        plsc.store_scatter(buf_v, [fb], pos_s)
        plsc.store_scatter(buf_v, [fb + 1], sk)
        plsc.addupdate_scatter(cnt_v, [binv], ones)
        act0 = slot + 1 >= CAP
        fmax0 = jnp.max(jnp.where(act0, binv, -1))

        def w_cond(c):
            return c[0] >= 0

        def w_body(c):
            b, act = c
            flush(b)
            act2 = jnp.logical_and(act, binv != b)
            return jnp.max(jnp.where(act2, binv, -1)), act2

        lax.while_loop(w_cond, w_body, (fmax0, act0))
        return carry

    lax.fori_loop(0, TSL // 16, body, 0)

    sign = jnp.int32(-2147483648)
    for b in range(NB):
        cb = _extract32(cnt_v, b)

        @pl.when(cb > 0)
        def _(b=b, cb=cb):
            eoff = i16 * 0 + (b * 2 * BUFW + 2 * (cb - 1))
            lp = plsc.load_gather(buf_v, [eoff])
            li = lax.bitwise_or(plsc.load_gather(buf_v, [eoff + 1]), sign)
            for g in range(2 * CAP // 16):
                sl = buf_v[pl.ds(b * 2 * BUFW + 16 * g, 16)]
                gpos = i16 + 16 * g
                ent = lax.shift_right_logical(gpos, 1)
                padv = jnp.where(lax.rem(gpos, 2) == 1, li, lp)
                buf_v[pl.ds(b * 2 * BUFW + 16 * g, 16)] = jnp.where(
                    ent >= cb, padv, sl)
            ofs = _extract32(gcur_v, b)
            dst = pl.multiple_of(2 * ofs, 2 * CAP)
            pltpu.sync_copy(buf_v.at[pl.ds(b * 2 * BUFW, 2 * CAP)],
                            binned_hbm.at[pl.ds(dst, 2 * CAP)])


def _k3_body(mem_hbm, val_hbm, binned_hbm, starts_hbm, out_hbm,
             sv, win_v, pos_v, lidx_v, pos_t, lidx_t, vrows_v, orows_v,
             accum, sem, sem2):
    cid = lax.axis_index("c")
    sid = lax.axis_index("s")
    i16 = _iota()
    pltpu.sync_copy(starts_hbm, sv)
    s0 = sv[pl.ds(0, 16)]
    s1 = sv[pl.ds(16, 16)]
    s2 = sv[pl.ds(32, 16)]

    def sget(k):
        v = (jnp.where(i16 == k, s0, 0) + jnp.where(i16 + 16 == k, s1, 0)
             + jnp.where(i16 + 32 == k, s2, 0))
        return jnp.sum(v)

    def stage_entries(eo, nent, base, for_accum):
        src = pl.multiple_of(2 * eo, 128)
        pltpu.sync_copy(binned_hbm.at[pl.ds(src, 2 * nent)],
                        win_v.at[pl.ds(0, 2 * nent)])
        mask31 = jnp.int32(2147483647)
        for g in range(nent // 16):
            e2 = 32 * g + 2 * i16
            p = plsc.load_gather(win_v, [e2])
            q = plsc.load_gather(win_v, [e2 + 1])
            l = lax.bitwise_and(q, mask31) - base
            if for_accum:
                l = jnp.where(q >= 0, l, RB)
            if nent == 64:
                pos_t[pl.ds(16 * g, 16)] = p
                lidx_t[pl.ds(16 * g, 16)] = l
            else:
                pos_v[g // 8, pl.ds(16 * (g % 8), 16)] = p
                lidx_v[g // 8, pl.ds(16 * (g % 8), 16)] = l

    def idx_refs(nent):
        if nent == 64:
            return [(pos_t, lidx_t, 0)]
        return [(pos_v.at[j], lidx_v.at[j], 128 * j) for j in range(4)]

    def accum_chunk(eo, nent, base):
        stage_entries(eo, nent, base, True)
        sub = min(nent, 128)
        refs = idx_refs(nent)
        gets = [pltpu.async_copy(
            val_hbm.at[pr], vrows_v.at[pl.ds(o, sub)], sem)
            for pr, _, o in refs]
        adds = []
        for g, (_, lr, o) in zip(gets, refs):
            g.wait()
            adds.append(pltpu.async_copy(
                vrows_v.at[pl.ds(o, sub)], accum.at[lr], sem2, add=True))
        for c in adds:
            c.wait()

    def out_chunk(eo, nent, base):
        stage_entries(eo, nent, base, False)
        sub = min(nent, 128)
        refs = idx_refs(nent)
        gets = [pltpu.async_copy(
            accum.at[lr], orows_v.at[pl.ds(o, sub)], sem)
            for _, lr, o in refs]
        puts = []
        for g, (pr, _, o) in zip(gets, refs):
            g.wait()
            puts.append(pltpu.async_copy(
                orows_v.at[pl.ds(o, sub)], out_hbm.at[pr], sem2))
        for c in puts:
            c.wait()

    def sweep(lo, hi, S, base, chunk):
        def m_cond(u):
            return u + 8 <= hi

        def m_body(u):
            chunk(S + 64 * u, 512, base)
            return u + 8

        u = lax.while_loop(m_cond, m_body, lo)

        def t_cond(u):
            return u < hi

        def t_body(u):
            chunk(S + 64 * u, 64, base)
            return u + 1

        lax.while_loop(t_cond, t_body, u)

    def bin_body(h, carry):
        b = cid * 16 + h
        S = sget(b)
        E = sget(b + 1)
        base = b * RB
        pltpu.sync_copy(mem_hbm.at[pl.ds(base + sid * (RB // 16), RB // 16)],
                        accum.at[pl.ds(sid * (RB // 16), RB // 16)])
        plsc.subcore_barrier()
        nb64 = lax.shift_right_logical(E - S, 6)
        share = lax.shift_right_logical(nb64 + 15, 4)
        lo = jnp.minimum(sid * share, nb64)
        hi = jnp.minimum(lo + share, nb64)
        sweep(lo, hi, S, base, accum_chunk)
        plsc.subcore_barrier()
        sweep(lo, hi, S, base, out_chunk)
        plsc.subcore_barrier()
        return carry

    lax.fori_loop(0, 16, bin_body, 0)


@functools.cache
def _get_kernels():
    mesh = plsc.VectorSubcoreMesh(core_axis_name="c", subcore_axis_name="s")
    k1 = functools.partial(
        pl.kernel, mesh=mesh, compiler_params=_cp,
        out_type=jax.ShapeDtypeStruct((NT, NB), jnp.int32),
        scratch_types=[pltpu.VMEM((TSL,), jnp.int32),
                       pltpu.VMEM((NB,), jnp.int32)],
    )(_k1_body)
    k2 = functools.partial(
        pl.kernel, mesh=mesh, compiler_params=_cp,
        out_type=jax.ShapeDtypeStruct((2 * BIN_CAP,), jnp.int32),
        scratch_types=[pltpu.VMEM((TSL,), jnp.int32),
                       pltpu.VMEM((NB * 2 * BUFW,), jnp.int32),
                       pltpu.VMEM((NB,), jnp.int32),
                       pltpu.VMEM((NB,), jnp.int32)],
    )(_k2_body)
    k3 = functools.partial(
        pl.kernel, mesh=mesh, compiler_params=_cp,
        out_type=jax.ShapeDtypeStruct((N, F), jnp.float32),
        scratch_types=[pltpu.VMEM((48,), jnp.int32),
                       pltpu.VMEM((1024,), jnp.int32),
                       pltpu.VMEM((4, 128), jnp.int32),
                       pltpu.VMEM((4, 128), jnp.int32),
                       pltpu.VMEM((64,), jnp.int32),
                       pltpu.VMEM((64,), jnp.int32),
                       pltpu.VMEM((512, F), jnp.float32),
                       pltpu.VMEM((512, F), jnp.float32),
                       pltpu.VMEM_SHARED((RB + 16, F), jnp.float32),
                       pltpu.SemaphoreType.DMA,
                       pltpu.SemaphoreType.DMA],
    )(_k3_body)
    return k1, k2, k3


def kernel(memory, idx, values):
    k1, k2, k3 = _get_kernels()
    counts = k1(idx)
    padded = ((counts + (CAP - 1)) // CAP) * CAP
    seg = padded.T.reshape(-1)
    ends = jnp.cumsum(seg)
    starts_flat = (ends - seg).astype(jnp.int32)
    gstart = starts_flat.reshape(NB, NT).T.astype(jnp.int32)
    bin_starts = jnp.concatenate(
        [starts_flat.reshape(NB, NT)[:, 0], ends[-1:].astype(jnp.int32)])
    starts48 = jnp.zeros((48,), jnp.int32).at[0:33].set(bin_starts)
    binned = k2(idx, gstart)
    out = k3(memory, values, binned, starts48)
    return out
